# baseline (speedup 1.0000x reference)
import functools

import jax
import jax.numpy as jnp
from jax import lax
from jax.experimental import pallas as pl
from jax.experimental.pallas import tpu as pltpu
from jax.experimental.pallas import tpu_sc as plsc

N = 8192
DIM = 2048
HID = 1536
E = 8
BLKR = 512
BLKG = 256
NPAD = 18432
NPADT = NPAD + N
NBG1 = NPAD // BLKG
NBT = NPADT // BLKG
NW = 32
TPW = N // NW
CH = 16


def _router_body(x_ref, wg_ref, tri_ref, i1_ref, i2_ref, r1_ref,
                 r2_ref, w1_ref, w2_ref, cnt_ref, run_ref):
    pid = pl.program_id(0)
    xb = x_ref[...]
    xbf = xb.astype(jnp.bfloat16)

    scores = lax.dot_general(
        xbf, wg_ref[...].astype(jnp.bfloat16), (((1,), (1,)), ((), ())),
        preferred_element_type=jnp.float32)
    smax = jnp.max(scores, axis=-1, keepdims=True)
    ex = jnp.exp(scores - smax)
    probs = ex / jnp.sum(ex, axis=-1, keepdims=True)
    idx8 = lax.broadcasted_iota(jnp.int32, (BLKR, E), 1)
    m1 = jnp.max(probs, axis=-1, keepdims=True)
    i1 = jnp.min(jnp.where(probs == m1, idx8, E), axis=-1, keepdims=True)
    probs2 = jnp.where(idx8 == i1, -jnp.inf, probs)
    m2 = jnp.max(probs2, axis=-1, keepdims=True)
    i2 = jnp.min(jnp.where(probs2 == m2, idx8, E), axis=-1, keepdims=True)
    wsum = m1 + m2 + 1e-9
    i1_ref[...] = i1
    i2_ref[...] = i2
    w1_ref[...] = m1 / wsum
    w2_ref[...] = m2 / wsum

    oh1 = (idx8 == i1).astype(jnp.int8)
    oh2 = (idx8 == i2).astype(jnp.int8)
    oh = jnp.concatenate([oh1, oh2], axis=0)
    excl = lax.dot_general(tri_ref[...], oh, (((1,), (0,)), ((), ())),
                           preferred_element_type=jnp.int32)

    @pl.when(pid == 0)
    def _():
        run_ref[...] = jnp.zeros_like(run_ref)

    run = run_ref[...]
    tb = excl + run
    r1_ref[...] = jnp.sum(tb[:BLKR] * oh1.astype(jnp.int32), axis=-1,
                          keepdims=True)
    r2_ref[...] = jnp.sum(tb[BLKR:] * oh2.astype(jnp.int32), axis=-1,
                          keepdims=True)
    newrun = run + jnp.sum(oh.astype(jnp.int32), axis=0, keepdims=True)
    run_ref[...] = newrun
    cnt_ref[...] = newrun


def _router(xf, Wg, tri, interpret=False):
    nb = N // BLKR
    outs = pl.pallas_call(
        _router_body,
        grid=(nb,),
        in_specs=[
            pl.BlockSpec((BLKR, DIM), lambda i: (i, 0)),
            pl.BlockSpec((E, DIM), lambda i: (0, 0)),
            pl.BlockSpec((2 * BLKR, 2 * BLKR), lambda i: (0, 0)),
        ],
        out_specs=[
            pl.BlockSpec((BLKR, 1), lambda i: (i, 0)),
            pl.BlockSpec((BLKR, 1), lambda i: (i, 0)),
            pl.BlockSpec((BLKR, 1), lambda i: (i, 0)),
            pl.BlockSpec((BLKR, 1), lambda i: (i, 0)),
            pl.BlockSpec((BLKR, 1), lambda i: (i, 0)),
            pl.BlockSpec((BLKR, 1), lambda i: (i, 0)),
            pl.BlockSpec((1, E), lambda i: (0, 0)),
        ],
        out_shape=[
            jax.ShapeDtypeStruct((N, 1), jnp.int32),
            jax.ShapeDtypeStruct((N, 1), jnp.int32),
            jax.ShapeDtypeStruct((N, 1), jnp.int32),
            jax.ShapeDtypeStruct((N, 1), jnp.int32),
            jax.ShapeDtypeStruct((N, 1), jnp.float32),
            jax.ShapeDtypeStruct((N, 1), jnp.float32),
            jax.ShapeDtypeStruct((1, E), jnp.int32),
        ],
        scratch_shapes=[pltpu.VMEM((1, E), jnp.int32)],
        compiler_params=pltpu.CompilerParams(
            dimension_semantics=("arbitrary",)),
        interpret=interpret,
    )(xf, Wg, tri)
    return outs


def _dest_body(i1_ref, i2_ref, r1_ref, r2_ref, pb_ref, d0_ref, d1_ref):
    blk = i1_ref.shape[0]
    idx16 = lax.broadcasted_iota(jnp.int32, (blk, 16), 1)
    pb = pb_ref[...]
    d0_ref[...] = jnp.sum(jnp.where(i1_ref[...] == idx16, pb, 0), axis=-1,
                          keepdims=True) + r1_ref[...]
    d1_ref[...] = jnp.sum(jnp.where(i2_ref[...] == idx16, pb, 0), axis=-1,
                          keepdims=True) + r2_ref[...]


def _dest(i1, i2, r1, r2, padbase, interpret=False):
    blk = min(1024, N)
    nb = N // blk
    return pl.pallas_call(
        _dest_body,
        grid=(nb,),
        in_specs=[
            pl.BlockSpec((blk, 1), lambda i: (i, 0)),
            pl.BlockSpec((blk, 1), lambda i: (i, 0)),
            pl.BlockSpec((blk, 1), lambda i: (i, 0)),
            pl.BlockSpec((blk, 1), lambda i: (i, 0)),
            pl.BlockSpec((1, 16), lambda i: (0, 0)),
        ],
        out_specs=[
            pl.BlockSpec((blk, 1), lambda i: (i, 0)),
            pl.BlockSpec((blk, 1), lambda i: (i, 0)),
        ],
        out_shape=[
            jax.ShapeDtypeStruct((N, 1), jnp.int32),
            jax.ShapeDtypeStruct((N, 1), jnp.int32),
        ],
        compiler_params=pltpu.CompilerParams(
            dimension_semantics=("parallel",)),
        interpret=interpret,
    )(i1, i2, r1, r2, padbase)


def _sc_dispatch_body(xfr, d0r, d1r, w1r, w2r, xg, wslot,
                      rb0, rb1, da0, da1, da2, db0, db1, db2, wa, wb,
                      sl0, sl1, sa0, sa1, sb0, sb1, swa0, swa1, swb0, swb1,
                      sda0, sda1, sda2, sdb0, sdb1, sdb2):
    wid = lax.axis_index("s") * 2 + lax.axis_index("c")
    tb = wid * TPW
    nch = TPW // CH
    pltpu.sync_copy(w1r.at[pl.ds(tb, TPW)], wa)
    pltpu.sync_copy(w2r.at[pl.ds(tb, TPW)], wb)
    rb = (rb0, rb1)
    da = (da0, da1, da2)
    db = (db0, db1, db2)
    lsem = (sl0, sl1)
    asem = (sa0, sa1)
    bsem = (sb0, sb1)
    wasem = (swa0, swa1)
    wbsem = (swb0, swb1)
    dasem = (sda0, sda1, sda2)
    dbsem = (sdb0, sdb1, sdb2)
    hl = [None, None]
    hA = [None, None]
    hB = [None, None]
    hWa = [None, None]
    hWb = [None, None]
    hDa = [None, None, None]
    hDb = [None, None, None]

    for c in range(min(3, nch)):
        hDa[c] = pltpu.async_copy(d0r.at[pl.ds(tb + c * CH, CH)], da[c],
                                  dasem[c])
        hDb[c] = pltpu.async_copy(d1r.at[pl.ds(tb + c * CH, CH)], db[c],
                                  dbsem[c])
    hl[0] = pltpu.async_copy(xfr.at[pl.ds(tb, CH)], rb[0], lsem[0])

    for c in range(nch):
        p = c & 1
        m = c % 3
        o = c * CH
        hl[p].wait()
        hDa[m].wait()
        hDb[m].wait()
        hA[p] = pltpu.async_copy(rb[p], xg.at[da[m]], asem[p])
        hB[p] = pltpu.async_copy(rb[p], xg.at[db[m]], bsem[p])
        hWa[p] = pltpu.async_copy(wa.at[pl.ds(o, CH)], wslot.at[da[m]],
                                  wasem[p])
        hWb[p] = pltpu.async_copy(wb.at[pl.ds(o, CH)], wslot.at[db[m]],
                                  wbsem[p])
        if c + 1 < nch:
            q = 1 - p
            if c >= 1:
                hA[q].wait()
                hB[q].wait()
                hWa[q].wait()
                hWb[q].wait()
                if c + 2 < nch:
                    m2 = (c + 2) % 3
                    hDa[m2] = pltpu.async_copy(
                        d0r.at[pl.ds(tb + (c + 2) * CH, CH)], da[m2],
                        dasem[m2])
                    hDb[m2] = pltpu.async_copy(
                        d1r.at[pl.ds(tb + (c + 2) * CH, CH)], db[m2],
                        dbsem[m2])
            hl[q] = pltpu.async_copy(xfr.at[pl.ds(tb + o + CH, CH)], rb[q],
                                     lsem[q])
    for p in (0, 1):
        if nch > p:
            hA[p].wait()
            hB[p].wait()
            hWa[p].wait()
            hWb[p].wait()


def _sc_dispatch(xf, d0, d1, w1, w2):
    mesh = plsc.VectorSubcoreMesh(core_axis_name="c", subcore_axis_name="s")
    fn = pl.kernel(
        _sc_dispatch_body,
        out_type=[
            jax.ShapeDtypeStruct((NPAD, DIM), jnp.float32),
            jax.ShapeDtypeStruct((NPAD,), jnp.float32),
        ],
        mesh=mesh,
        scratch_types=(
            [pltpu.VMEM((CH, DIM), jnp.float32)] * 2
            + [pltpu.VMEM((CH,), jnp.int32)] * 6
            + [pltpu.VMEM((TPW,), jnp.float32)] * 2
            + [pltpu.SemaphoreType.DMA] * 16
        ),
    )
    return fn(xf, d0, d1, w1, w2)


def _ffn_body(be_ref, xg_ref, xf_ref, w1_ref, w3_ref, w2_ref, ws_ref, o_ref):
    j = pl.program_id(0)
    k = pl.program_id(1)
    is_sh = j >= NBG1
    xb = jnp.where(is_sh, xf_ref[...], xg_ref[...])
    h1 = lax.dot_general(xb, w1_ref[0], (((1,), (1,)), ((), ())),
                         preferred_element_type=jnp.float32)
    h3 = lax.dot_general(xb, w3_ref[0], (((1,), (1,)), ((), ())),
                         preferred_element_type=jnp.float32)
    h = h1 * jax.nn.sigmoid(h1) * h3
    y = lax.dot_general(h, w2_ref[0], (((1,), (1,)), ((), ())),
                        preferred_element_type=jnp.float32)
    ws = jnp.where(is_sh, jnp.ones_like(ws_ref[...]), ws_ref[...])
    yw = y * ws

    @pl.when(k == 0)
    def _():
        o_ref[...] = yw

    @pl.when(k == 1)
    def _():
        o_ref[...] += yw


def _ffn(block_expert, xg, xf, W1c, W3c, W2c, wslot2d, interpret=False):
    hid2 = HID // 2
    grid_spec = pltpu.PrefetchScalarGridSpec(
        num_scalar_prefetch=1,
        grid=(NBT, 2),
        in_specs=[
            pl.BlockSpec((BLKG, DIM),
                         lambda j, k, be: (jnp.minimum(j, NBG1 - 1), 0)),
            pl.BlockSpec((BLKG, DIM),
                         lambda j, k, be: (jnp.maximum(j - NBG1, 0), 0)),
            pl.BlockSpec((1, hid2, DIM), lambda j, k, be: (be[j], k, 0)),
            pl.BlockSpec((1, hid2, DIM), lambda j, k, be: (be[j], k, 0)),
            pl.BlockSpec((1, DIM, hid2), lambda j, k, be: (be[j], 0, k)),
            pl.BlockSpec((BLKG, 1),
                         lambda j, k, be: (jnp.minimum(j, NBG1 - 1), 0)),
        ],
        out_specs=pl.BlockSpec((BLKG, DIM), lambda j, k, be: (j, 0)),
    )
    return pl.pallas_call(
        _ffn_body,
        grid_spec=grid_spec,
        out_shape=jax.ShapeDtypeStruct((NPADT, DIM), jnp.float32),
        compiler_params=pltpu.CompilerParams(
            dimension_semantics=("arbitrary", "arbitrary")),
        interpret=interpret,
    )(block_expert, xg, xf, W1c, W3c, W2c, wslot2d)


def _sc_gather_body(outg, d0r, d1r, g0, g1,
                    rb0, rb1, di0, di1, di2,
                    sg0, sg1, sw0, sw1, sd0, sd1, sd2):
    wid = lax.axis_index("s") * 2 + lax.axis_index("c")
    tb = wid * TPW
    nt = 2 * (TPW // CH)
    rb = (rb0, rb1)
    di = (di0, di1, di2)
    sg = (sg0, sg1)
    sw = (sw0, sw1)
    sd = (sd0, sd1, sd2)
    hG = [None, None]
    hW = [None, None]
    hD = [None, None, None]

    def idx_src(t):
        r = d0r if t % 2 == 0 else d1r
        return r.at[pl.ds(tb + (t >> 1) * CH, CH)]

    def out_dst(t):
        r = g0 if t % 2 == 0 else g1
        return r.at[pl.ds(tb + (t >> 1) * CH, CH)]

    for t in range(min(3, nt)):
        hD[t] = pltpu.async_copy(idx_src(t), di[t], sd[t])

    for t in range(nt):
        p = t & 1
        m = t % 3
        if t >= 2:
            hW[p].wait()
        hD[m].wait()
        hG[p] = pltpu.async_copy(outg.at[di[m]], rb[p], sg[p])
        if t >= 1:
            q = 1 - p
            hG[q].wait()
            if t + 2 < nt:
                m2 = (t + 2) % 3
                hD[m2] = pltpu.async_copy(idx_src(t + 2), di[m2], sd[m2])
            hW[q] = pltpu.async_copy(rb[q], out_dst(t - 1), sw[q])
    p = (nt - 1) & 1
    hG[p].wait()
    hW[p] = pltpu.async_copy(rb[p], out_dst(nt - 1), sw[p])
    hW[0].wait()
    hW[1].wait()


def _sc_gather(outg, d0r, d1r):
    mesh = plsc.VectorSubcoreMesh(core_axis_name="c", subcore_axis_name="s")
    fn = pl.kernel(
        _sc_gather_body,
        out_type=[
            jax.ShapeDtypeStruct((N, DIM), jnp.float32),
            jax.ShapeDtypeStruct((N, DIM), jnp.float32),
        ],
        mesh=mesh,
        scratch_types=(
            [pltpu.VMEM((CH, DIM), jnp.float32)] * 2
            + [pltpu.VMEM((CH,), jnp.int32)] * 3
            + [pltpu.SemaphoreType.DMA] * 7
        ),
    )
    return fn(outg, d0r, d1r)


def _combine_body(g0_ref, g1_ref, sh_ref, o_ref):
    o_ref[...] = g0_ref[...] + g1_ref[...] + sh_ref[...]


def _combine(g0, g1, outg, interpret=False):
    blk = min(512, N)
    nb = N // blk
    base = NPAD // blk
    return pl.pallas_call(
        _combine_body,
        grid=(nb,),
        in_specs=[
            pl.BlockSpec((blk, DIM), lambda i: (i, 0)),
            pl.BlockSpec((blk, DIM), lambda i: (i, 0)),
            pl.BlockSpec((blk, DIM), lambda i: (base + i, 0)),
        ],
        out_specs=pl.BlockSpec((blk, DIM), lambda i: (i, 0)),
        out_shape=jax.ShapeDtypeStruct((N, DIM), jnp.float32),
        compiler_params=pltpu.CompilerParams(
            dimension_semantics=("parallel",)),
        interpret=interpret,
    )(g0, g1, outg)


def _metadata(counts):
    cnt_pad = ((counts + BLKG - 1) // BLKG) * BLKG
    padbase = jnp.concatenate(
        [jnp.zeros((1,), jnp.int32), jnp.cumsum(cnt_pad)[:-1].astype(jnp.int32),
         jnp.zeros((16 - E,), jnp.int32)])
    off = jnp.arange(NBG1, dtype=jnp.int32) * BLKG
    be1 = (jnp.sum((padbase[None, :E] <= off[:, None]).astype(jnp.int32),
                   axis=1) - 1).astype(jnp.int32)
    block_expert = jnp.concatenate(
        [be1, jnp.full((NBT - NBG1,), E, jnp.int32)])
    return padbase, block_expert


def kernel(x, Wg, W1, W2, W3, W1s, W2s, W3s):
    bsz, seqlen, dim = x.shape
    xf = x.reshape(-1, dim)
    ar = jnp.arange(2 * BLKR, dtype=jnp.int32)
    tri = (ar[:, None] > ar[None, :]).astype(jnp.int8)
    W1c = jnp.concatenate([W1, W1s[None]], 0)
    W3c = jnp.concatenate([W3, W3s[None]], 0)
    W2c = jnp.concatenate([W2, W2s[None]], 0)

    i1, i2, r1, r2, w1, w2, cnt = _router(xf, Wg, tri)
    counts = cnt[0]
    padbase, block_expert = _metadata(counts)

    d0, d1 = _dest(i1, i2, r1, r2, padbase.reshape(1, 16))
    dest0 = d0.reshape(N)
    dest1 = d1.reshape(N)
    xg, wslot = _sc_dispatch(xf, dest0, dest1, w1.reshape(N), w2.reshape(N))

    outg = _ffn(block_expert, xg, xf, W1c, W3c, W2c, wslot.reshape(NPAD, 1))

    g0, g1 = _sc_gather(outg, dest0, dest1)
    y = _combine(g0, g1, outg)
    return y.reshape(bsz, seqlen, dim)


# speedup vs baseline: 1.7515x; 1.7515x over previous
import functools

import jax
import jax.numpy as jnp
from jax import lax
from jax.experimental import pallas as pl
from jax.experimental.pallas import tpu as pltpu
from jax.experimental.pallas import tpu_sc as plsc

N = 8192
DIM = 2048
HID = 1536
E = 8
BLKR = 512
BLKG = 256
NPAD = 18432
NPADT = NPAD + N
NBG1 = NPAD // BLKG
NBT = NPADT // BLKG
NW = 32
TPW = N // NW
CH = 16


def _router_body(x_ref, wg_ref, tri_ref, i1_ref, i2_ref, r1_ref,
                 r2_ref, w1_ref, w2_ref, cnt_ref, run_ref):
    pid = pl.program_id(0)
    xb = x_ref[...]
    xbf = xb.astype(jnp.bfloat16)

    scores = lax.dot_general(
        xbf, wg_ref[...].astype(jnp.bfloat16), (((1,), (1,)), ((), ())),
        preferred_element_type=jnp.float32)
    smax = jnp.max(scores, axis=-1, keepdims=True)
    ex = jnp.exp(scores - smax)
    probs = ex / jnp.sum(ex, axis=-1, keepdims=True)
    idx8 = lax.broadcasted_iota(jnp.int32, (BLKR, E), 1)
    m1 = jnp.max(probs, axis=-1, keepdims=True)
    i1 = jnp.min(jnp.where(probs == m1, idx8, E), axis=-1, keepdims=True)
    probs2 = jnp.where(idx8 == i1, -jnp.inf, probs)
    m2 = jnp.max(probs2, axis=-1, keepdims=True)
    i2 = jnp.min(jnp.where(probs2 == m2, idx8, E), axis=-1, keepdims=True)
    wsum = m1 + m2 + 1e-9
    i1_ref[...] = i1
    i2_ref[...] = i2
    w1_ref[...] = m1 / wsum
    w2_ref[...] = m2 / wsum

    oh1 = (idx8 == i1).astype(jnp.int8)
    oh2 = (idx8 == i2).astype(jnp.int8)
    oh = jnp.concatenate([oh1, oh2], axis=0)
    excl = lax.dot_general(tri_ref[...], oh, (((1,), (0,)), ((), ())),
                           preferred_element_type=jnp.int32)

    @pl.when(pid == 0)
    def _():
        run_ref[...] = jnp.zeros_like(run_ref)

    run = run_ref[...]
    tb = excl + run
    r1_ref[...] = jnp.sum(tb[:BLKR] * oh1.astype(jnp.int32), axis=-1,
                          keepdims=True)
    r2_ref[...] = jnp.sum(tb[BLKR:] * oh2.astype(jnp.int32), axis=-1,
                          keepdims=True)
    newrun = run + jnp.sum(oh.astype(jnp.int32), axis=0, keepdims=True)
    run_ref[...] = newrun
    cnt_ref[...] = newrun


def _router(xf, Wg, tri, interpret=False):
    nb = N // BLKR
    outs = pl.pallas_call(
        _router_body,
        grid=(nb,),
        in_specs=[
            pl.BlockSpec((BLKR, DIM), lambda i: (i, 0)),
            pl.BlockSpec((E, DIM), lambda i: (0, 0)),
            pl.BlockSpec((2 * BLKR, 2 * BLKR), lambda i: (0, 0)),
        ],
        out_specs=[
            pl.BlockSpec((BLKR, 1), lambda i: (i, 0)),
            pl.BlockSpec((BLKR, 1), lambda i: (i, 0)),
            pl.BlockSpec((BLKR, 1), lambda i: (i, 0)),
            pl.BlockSpec((BLKR, 1), lambda i: (i, 0)),
            pl.BlockSpec((BLKR, 1), lambda i: (i, 0)),
            pl.BlockSpec((BLKR, 1), lambda i: (i, 0)),
            pl.BlockSpec((1, E), lambda i: (0, 0)),
        ],
        out_shape=[
            jax.ShapeDtypeStruct((N, 1), jnp.int32),
            jax.ShapeDtypeStruct((N, 1), jnp.int32),
            jax.ShapeDtypeStruct((N, 1), jnp.int32),
            jax.ShapeDtypeStruct((N, 1), jnp.int32),
            jax.ShapeDtypeStruct((N, 1), jnp.float32),
            jax.ShapeDtypeStruct((N, 1), jnp.float32),
            jax.ShapeDtypeStruct((1, E), jnp.int32),
        ],
        scratch_shapes=[pltpu.VMEM((1, E), jnp.int32)],
        compiler_params=pltpu.CompilerParams(
            dimension_semantics=("arbitrary",)),
        interpret=interpret,
    )(xf, Wg, tri)
    return outs


def _dest_body(i1_ref, i2_ref, r1_ref, r2_ref, pb_ref, d0_ref, d1_ref):
    blk = i1_ref.shape[0]
    idx16 = lax.broadcasted_iota(jnp.int32, (blk, 16), 1)
    pb = pb_ref[...]
    d0_ref[...] = jnp.sum(jnp.where(i1_ref[...] == idx16, pb, 0), axis=-1,
                          keepdims=True) + r1_ref[...]
    d1_ref[...] = jnp.sum(jnp.where(i2_ref[...] == idx16, pb, 0), axis=-1,
                          keepdims=True) + r2_ref[...]


def _dest(i1, i2, r1, r2, padbase, interpret=False):
    blk = min(1024, N)
    nb = N // blk
    return pl.pallas_call(
        _dest_body,
        grid=(nb,),
        in_specs=[
            pl.BlockSpec((blk, 1), lambda i: (i, 0)),
            pl.BlockSpec((blk, 1), lambda i: (i, 0)),
            pl.BlockSpec((blk, 1), lambda i: (i, 0)),
            pl.BlockSpec((blk, 1), lambda i: (i, 0)),
            pl.BlockSpec((1, 16), lambda i: (0, 0)),
        ],
        out_specs=[
            pl.BlockSpec((blk, 1), lambda i: (i, 0)),
            pl.BlockSpec((blk, 1), lambda i: (i, 0)),
        ],
        out_shape=[
            jax.ShapeDtypeStruct((N, 1), jnp.int32),
            jax.ShapeDtypeStruct((N, 1), jnp.int32),
        ],
        compiler_params=pltpu.CompilerParams(
            dimension_semantics=("parallel",)),
        interpret=interpret,
    )(i1, i2, r1, r2, padbase)


def _sc_dispatch_body(xfr, d0r, d1r, w1r, w2r, xg, wslot,
                      rb0, rb1, da0, da1, da2, db0, db1, db2, wa, wb,
                      sl0, sl1, sa0, sa1, sb0, sb1, swa0, swa1, swb0, swb1,
                      sda0, sda1, sda2, sdb0, sdb1, sdb2):
    wid = lax.axis_index("s") * 2 + lax.axis_index("c")
    tb = wid * TPW
    nch = TPW // CH
    pltpu.sync_copy(w1r.at[pl.ds(tb, TPW)], wa)
    pltpu.sync_copy(w2r.at[pl.ds(tb, TPW)], wb)
    rb = (rb0, rb1)
    da = (da0, da1, da2)
    db = (db0, db1, db2)
    lsem = (sl0, sl1)
    asem = (sa0, sa1)
    bsem = (sb0, sb1)
    wasem = (swa0, swa1)
    wbsem = (swb0, swb1)
    dasem = (sda0, sda1, sda2)
    dbsem = (sdb0, sdb1, sdb2)
    hl = [None, None]
    hA = [None, None]
    hB = [None, None]
    hWa = [None, None]
    hWb = [None, None]
    hDa = [None, None, None]
    hDb = [None, None, None]

    for c in range(min(3, nch)):
        hDa[c] = pltpu.async_copy(d0r.at[pl.ds(tb + c * CH, CH)], da[c],
                                  dasem[c])
        hDb[c] = pltpu.async_copy(d1r.at[pl.ds(tb + c * CH, CH)], db[c],
                                  dbsem[c])
    hl[0] = pltpu.async_copy(xfr.at[pl.ds(tb, CH)], rb[0], lsem[0])

    for c in range(nch):
        p = c & 1
        m = c % 3
        o = c * CH
        hl[p].wait()
        hDa[m].wait()
        hDb[m].wait()
        hA[p] = pltpu.async_copy(rb[p], xg.at[da[m]], asem[p])
        hB[p] = pltpu.async_copy(rb[p], xg.at[db[m]], bsem[p])
        hWa[p] = pltpu.async_copy(wa.at[pl.ds(o, CH)], wslot.at[da[m]],
                                  wasem[p])
        hWb[p] = pltpu.async_copy(wb.at[pl.ds(o, CH)], wslot.at[db[m]],
                                  wbsem[p])
        if c + 1 < nch:
            q = 1 - p
            if c >= 1:
                hA[q].wait()
                hB[q].wait()
                hWa[q].wait()
                hWb[q].wait()
                if c + 2 < nch:
                    m2 = (c + 2) % 3
                    hDa[m2] = pltpu.async_copy(
                        d0r.at[pl.ds(tb + (c + 2) * CH, CH)], da[m2],
                        dasem[m2])
                    hDb[m2] = pltpu.async_copy(
                        d1r.at[pl.ds(tb + (c + 2) * CH, CH)], db[m2],
                        dbsem[m2])
            hl[q] = pltpu.async_copy(xfr.at[pl.ds(tb + o + CH, CH)], rb[q],
                                     lsem[q])
    for p in (0, 1):
        if nch > p:
            hA[p].wait()
            hB[p].wait()
            hWa[p].wait()
            hWb[p].wait()


def _sc_dispatch(xf, d0, d1, w1, w2):
    mesh = plsc.VectorSubcoreMesh(core_axis_name="c", subcore_axis_name="s")
    fn = pl.kernel(
        _sc_dispatch_body,
        out_type=[
            jax.ShapeDtypeStruct((NPAD, DIM), jnp.float32),
            jax.ShapeDtypeStruct((NPAD,), jnp.float32),
        ],
        mesh=mesh,
        scratch_types=(
            [pltpu.VMEM((CH, DIM), jnp.float32)] * 2
            + [pltpu.VMEM((CH,), jnp.int32)] * 6
            + [pltpu.VMEM((TPW,), jnp.float32)] * 2
            + [pltpu.SemaphoreType.DMA] * 16
        ),
    )
    return fn(xf, d0, d1, w1, w2)


def _cast_body(w_ref, o_ref):
    o_ref[...] = w_ref[...].astype(jnp.bfloat16)


def _cast(W, interpret=False):
    e, a, b = W.shape
    return pl.pallas_call(
        _cast_body,
        grid=(e,),
        in_specs=[pl.BlockSpec((1, a, b), lambda i: (i, 0, 0))],
        out_specs=pl.BlockSpec((1, a, b), lambda i: (i, 0, 0)),
        out_shape=jax.ShapeDtypeStruct((e, a, b), jnp.bfloat16),
        compiler_params=pltpu.CompilerParams(
            dimension_semantics=("arbitrary",)),
        interpret=interpret,
    )(W)


def _ffn_body(be_ref, xg_ref, w1_ref, w3_ref, w2_ref, ws_ref, o_ref):
    e = be_ref[pl.program_id(0)]

    @pl.when(e >= 0)
    def _():
        xb = xg_ref[...].astype(jnp.bfloat16)
        h1 = lax.dot_general(xb, w1_ref[0], (((1,), (1,)), ((), ())),
                             preferred_element_type=jnp.float32)
        h3 = lax.dot_general(xb, w3_ref[0], (((1,), (1,)), ((), ())),
                             preferred_element_type=jnp.float32)
        h = (h1 * jax.nn.sigmoid(h1) * h3).astype(jnp.bfloat16)
        y = lax.dot_general(h, w2_ref[0], (((1,), (1,)), ((), ())),
                            preferred_element_type=jnp.float32)
        o_ref[...] = y * ws_ref[...]


def _ffn(block_expert, xg, W1b, W3b, W2b, wslot2d, interpret=False):
    grid_spec = pltpu.PrefetchScalarGridSpec(
        num_scalar_prefetch=1,
        grid=(NBG1,),
        in_specs=[
            pl.BlockSpec((BLKG, DIM), lambda j, be: (j, 0)),
            pl.BlockSpec((1, HID, DIM),
                         lambda j, be: (jnp.maximum(be[j], 0), 0, 0)),
            pl.BlockSpec((1, HID, DIM),
                         lambda j, be: (jnp.maximum(be[j], 0), 0, 0)),
            pl.BlockSpec((1, DIM, HID),
                         lambda j, be: (jnp.maximum(be[j], 0), 0, 0)),
            pl.BlockSpec((BLKG, 1), lambda j, be: (j, 0)),
        ],
        out_specs=pl.BlockSpec((BLKG, DIM), lambda j, be: (j, 0)),
    )
    return pl.pallas_call(
        _ffn_body,
        grid_spec=grid_spec,
        out_shape=jax.ShapeDtypeStruct((NPAD, DIM), jnp.float32),
        compiler_params=pltpu.CompilerParams(
            dimension_semantics=("arbitrary",)),
        interpret=interpret,
    )(block_expert, xg, W1b, W3b, W2b, wslot2d)


def _ffns_body(xf_ref, w1_ref, w3_ref, w2_ref, o_ref):
    xb = xf_ref[...].astype(jnp.bfloat16)
    h1 = lax.dot_general(xb, w1_ref[...], (((1,), (1,)), ((), ())),
                         preferred_element_type=jnp.float32)
    h3 = lax.dot_general(xb, w3_ref[...], (((1,), (1,)), ((), ())),
                         preferred_element_type=jnp.float32)
    h = (h1 * jax.nn.sigmoid(h1) * h3).astype(jnp.bfloat16)
    o_ref[...] = lax.dot_general(h, w2_ref[...], (((1,), (1,)), ((), ())),
                                 preferred_element_type=jnp.float32)


def _ffns(xf, W1sb, W3sb, W2sb, interpret=False):
    nb = N // BLKG
    return pl.pallas_call(
        _ffns_body,
        grid=(nb,),
        in_specs=[
            pl.BlockSpec((BLKG, DIM), lambda j: (j, 0)),
            pl.BlockSpec((HID, DIM), lambda j: (0, 0)),
            pl.BlockSpec((HID, DIM), lambda j: (0, 0)),
            pl.BlockSpec((DIM, HID), lambda j: (0, 0)),
        ],
        out_specs=pl.BlockSpec((BLKG, DIM), lambda j: (j, 0)),
        out_shape=jax.ShapeDtypeStruct((N, DIM), jnp.float32),
        compiler_params=pltpu.CompilerParams(
            dimension_semantics=("arbitrary",)),
        interpret=interpret,
    )(xf, W1sb, W3sb, W2sb)


def _sc_gather_body(outg, d0r, d1r, g0, g1,
                    rb0, rb1, di0, di1, di2,
                    sg0, sg1, sw0, sw1, sd0, sd1, sd2):
    wid = lax.axis_index("s") * 2 + lax.axis_index("c")
    tb = wid * TPW
    nt = 2 * (TPW // CH)
    rb = (rb0, rb1)
    di = (di0, di1, di2)
    sg = (sg0, sg1)
    sw = (sw0, sw1)
    sd = (sd0, sd1, sd2)
    hG = [None, None]
    hW = [None, None]
    hD = [None, None, None]

    def idx_src(t):
        r = d0r if t % 2 == 0 else d1r
        return r.at[pl.ds(tb + (t >> 1) * CH, CH)]

    def out_dst(t):
        r = g0 if t % 2 == 0 else g1
        return r.at[pl.ds(tb + (t >> 1) * CH, CH)]

    for t in range(min(3, nt)):
        hD[t] = pltpu.async_copy(idx_src(t), di[t], sd[t])

    for t in range(nt):
        p = t & 1
        m = t % 3
        if t >= 2:
            hW[p].wait()
        hD[m].wait()
        hG[p] = pltpu.async_copy(outg.at[di[m]], rb[p], sg[p])
        if t >= 1:
            q = 1 - p
            hG[q].wait()
            if t + 2 < nt:
                m2 = (t + 2) % 3
                hD[m2] = pltpu.async_copy(idx_src(t + 2), di[m2], sd[m2])
            hW[q] = pltpu.async_copy(rb[q], out_dst(t - 1), sw[q])
    p = (nt - 1) & 1
    hG[p].wait()
    hW[p] = pltpu.async_copy(rb[p], out_dst(nt - 1), sw[p])
    hW[0].wait()
    hW[1].wait()


def _sc_gather(outg, d0r, d1r):
    mesh = plsc.VectorSubcoreMesh(core_axis_name="c", subcore_axis_name="s")
    fn = pl.kernel(
        _sc_gather_body,
        out_type=[
            jax.ShapeDtypeStruct((N, DIM), jnp.float32),
            jax.ShapeDtypeStruct((N, DIM), jnp.float32),
        ],
        mesh=mesh,
        scratch_types=(
            [pltpu.VMEM((CH, DIM), jnp.float32)] * 2
            + [pltpu.VMEM((CH,), jnp.int32)] * 3
            + [pltpu.SemaphoreType.DMA] * 7
        ),
    )
    return fn(outg, d0r, d1r)


def _combine_body(g0_ref, g1_ref, sh_ref, o_ref):
    o_ref[...] = g0_ref[...] + g1_ref[...] + sh_ref[...]


def _combine(g0, g1, outs, interpret=False):
    blk = min(512, N)
    nb = N // blk
    return pl.pallas_call(
        _combine_body,
        grid=(nb,),
        in_specs=[
            pl.BlockSpec((blk, DIM), lambda i: (i, 0)),
            pl.BlockSpec((blk, DIM), lambda i: (i, 0)),
            pl.BlockSpec((blk, DIM), lambda i: (i, 0)),
        ],
        out_specs=pl.BlockSpec((blk, DIM), lambda i: (i, 0)),
        out_shape=jax.ShapeDtypeStruct((N, DIM), jnp.float32),
        compiler_params=pltpu.CompilerParams(
            dimension_semantics=("parallel",)),
        interpret=interpret,
    )(g0, g1, outs)


def _metadata(counts):
    cnt_pad = ((counts + BLKG - 1) // BLKG) * BLKG
    padbase = jnp.concatenate(
        [jnp.zeros((1,), jnp.int32), jnp.cumsum(cnt_pad)[:-1].astype(jnp.int32),
         jnp.zeros((16 - E,), jnp.int32)])
    off = jnp.arange(NBG1, dtype=jnp.int32) * BLKG
    be1 = (jnp.sum((padbase[None, :E] <= off[:, None]).astype(jnp.int32),
                   axis=1) - 1).astype(jnp.int32)
    total_pad = jnp.sum(cnt_pad)
    block_expert = jnp.where(off < total_pad, be1, -1)
    return padbase, block_expert


def kernel(x, Wg, W1, W2, W3, W1s, W2s, W3s):
    bsz, seqlen, dim = x.shape
    xf = x.reshape(-1, dim)
    ar = jnp.arange(2 * BLKR, dtype=jnp.int32)
    tri = (ar[:, None] > ar[None, :]).astype(jnp.int8)
    W1b = _cast(W1)
    W3b = _cast(W3)
    W2b = _cast(W2)
    W1sb = W1s.astype(jnp.bfloat16)
    W3sb = W3s.astype(jnp.bfloat16)
    W2sb = W2s.astype(jnp.bfloat16)

    i1, i2, r1, r2, w1, w2, cnt = _router(xf, Wg, tri)
    counts = cnt[0]
    padbase, block_expert = _metadata(counts)

    d0, d1 = _dest(i1, i2, r1, r2, padbase.reshape(1, 16))
    dest0 = d0.reshape(N)
    dest1 = d1.reshape(N)
    xg, wslot = _sc_dispatch(xf, dest0, dest1, w1.reshape(N), w2.reshape(N))

    outs = _ffns(xf, W1sb, W3sb, W2sb)
    outg = _ffn(block_expert, xg, W1b, W3b, W2b, wslot.reshape(NPAD, 1))

    g0, g1 = _sc_gather(outg, dest0, dest1)
    y = _combine(g0, g1, outs)
    return y.reshape(bsz, seqlen, dim)


# speedup vs baseline: 1.8612x; 1.0626x over previous
import functools

import jax
import jax.numpy as jnp
from jax import lax
from jax.experimental import pallas as pl
from jax.experimental.pallas import tpu as pltpu
from jax.experimental.pallas import tpu_sc as plsc

N = 8192
DIM = 2048
HID = 1536
E = 8
BLKR = 512
BLKG = 256
NPAD = 18432
NPADT = NPAD + N
NBG1 = NPAD // BLKG
NBT = NPADT // BLKG
NW = 32
TPW = N // NW
CH = 16


def _router_body(x_ref, wg_ref, tri_ref, i1_ref, i2_ref, r1_ref,
                 r2_ref, w1_ref, w2_ref, cnt_ref, run_ref):
    pid = pl.program_id(0)
    xb = x_ref[...]
    xbf = xb.astype(jnp.bfloat16)

    scores = lax.dot_general(
        xbf, wg_ref[...].astype(jnp.bfloat16), (((1,), (1,)), ((), ())),
        preferred_element_type=jnp.float32)
    smax = jnp.max(scores, axis=-1, keepdims=True)
    ex = jnp.exp(scores - smax)
    probs = ex / jnp.sum(ex, axis=-1, keepdims=True)
    idx8 = lax.broadcasted_iota(jnp.int32, (BLKR, E), 1)
    m1 = jnp.max(probs, axis=-1, keepdims=True)
    i1 = jnp.min(jnp.where(probs == m1, idx8, E), axis=-1, keepdims=True)
    probs2 = jnp.where(idx8 == i1, -jnp.inf, probs)
    m2 = jnp.max(probs2, axis=-1, keepdims=True)
    i2 = jnp.min(jnp.where(probs2 == m2, idx8, E), axis=-1, keepdims=True)
    wsum = m1 + m2 + 1e-9
    i1_ref[...] = i1
    i2_ref[...] = i2
    w1_ref[...] = m1 / wsum
    w2_ref[...] = m2 / wsum

    oh1 = (idx8 == i1).astype(jnp.int8)
    oh2 = (idx8 == i2).astype(jnp.int8)
    oh = jnp.concatenate([oh1, oh2], axis=0)
    excl = lax.dot_general(tri_ref[...], oh, (((1,), (0,)), ((), ())),
                           preferred_element_type=jnp.int32)

    @pl.when(pid == 0)
    def _():
        run_ref[...] = jnp.zeros_like(run_ref)

    run = run_ref[...]
    tb = excl + run
    r1_ref[...] = jnp.sum(tb[:BLKR] * oh1.astype(jnp.int32), axis=-1,
                          keepdims=True)
    r2_ref[...] = jnp.sum(tb[BLKR:] * oh2.astype(jnp.int32), axis=-1,
                          keepdims=True)
    newrun = run + jnp.sum(oh.astype(jnp.int32), axis=0, keepdims=True)
    run_ref[...] = newrun
    cnt_ref[...] = newrun


def _router(xf, Wg, tri, interpret=False):
    nb = N // BLKR
    outs = pl.pallas_call(
        _router_body,
        grid=(nb,),
        in_specs=[
            pl.BlockSpec((BLKR, DIM), lambda i: (i, 0)),
            pl.BlockSpec((E, DIM), lambda i: (0, 0)),
            pl.BlockSpec((2 * BLKR, 2 * BLKR), lambda i: (0, 0)),
        ],
        out_specs=[
            pl.BlockSpec((BLKR, 1), lambda i: (i, 0)),
            pl.BlockSpec((BLKR, 1), lambda i: (i, 0)),
            pl.BlockSpec((BLKR, 1), lambda i: (i, 0)),
            pl.BlockSpec((BLKR, 1), lambda i: (i, 0)),
            pl.BlockSpec((BLKR, 1), lambda i: (i, 0)),
            pl.BlockSpec((BLKR, 1), lambda i: (i, 0)),
            pl.BlockSpec((1, E), lambda i: (0, 0)),
        ],
        out_shape=[
            jax.ShapeDtypeStruct((N, 1), jnp.int32),
            jax.ShapeDtypeStruct((N, 1), jnp.int32),
            jax.ShapeDtypeStruct((N, 1), jnp.int32),
            jax.ShapeDtypeStruct((N, 1), jnp.int32),
            jax.ShapeDtypeStruct((N, 1), jnp.float32),
            jax.ShapeDtypeStruct((N, 1), jnp.float32),
            jax.ShapeDtypeStruct((1, E), jnp.int32),
        ],
        scratch_shapes=[pltpu.VMEM((1, E), jnp.int32)],
        compiler_params=pltpu.CompilerParams(
            dimension_semantics=("arbitrary",)),
        interpret=interpret,
    )(xf, Wg, tri)
    return outs


def _dest_body(i1_ref, i2_ref, r1_ref, r2_ref, pb_ref, d0_ref, d1_ref):
    blk = i1_ref.shape[0]
    idx16 = lax.broadcasted_iota(jnp.int32, (blk, 16), 1)
    pb = pb_ref[...]
    d0_ref[...] = jnp.sum(jnp.where(i1_ref[...] == idx16, pb, 0), axis=-1,
                          keepdims=True) + r1_ref[...]
    d1_ref[...] = jnp.sum(jnp.where(i2_ref[...] == idx16, pb, 0), axis=-1,
                          keepdims=True) + r2_ref[...]


def _dest(i1, i2, r1, r2, padbase, interpret=False):
    blk = min(1024, N)
    nb = N // blk
    return pl.pallas_call(
        _dest_body,
        grid=(nb,),
        in_specs=[
            pl.BlockSpec((blk, 1), lambda i: (i, 0)),
            pl.BlockSpec((blk, 1), lambda i: (i, 0)),
            pl.BlockSpec((blk, 1), lambda i: (i, 0)),
            pl.BlockSpec((blk, 1), lambda i: (i, 0)),
            pl.BlockSpec((1, 16), lambda i: (0, 0)),
        ],
        out_specs=[
            pl.BlockSpec((blk, 1), lambda i: (i, 0)),
            pl.BlockSpec((blk, 1), lambda i: (i, 0)),
        ],
        out_shape=[
            jax.ShapeDtypeStruct((N, 1), jnp.int32),
            jax.ShapeDtypeStruct((N, 1), jnp.int32),
        ],
        compiler_params=pltpu.CompilerParams(
            dimension_semantics=("parallel",)),
        interpret=interpret,
    )(i1, i2, r1, r2, padbase)


def _sc_dispatch_body(xfr, d0r, d1r, w1r, w2r, xg, wslot,
                      rb0, rb1, da0, da1, da2, db0, db1, db2, wa, wb,
                      sl0, sl1, sa0, sa1, sb0, sb1, swa0, swa1, swb0, swb1,
                      sda0, sda1, sda2, sdb0, sdb1, sdb2):
    wid = lax.axis_index("s") * 2 + lax.axis_index("c")
    tb = wid * TPW
    nch = TPW // CH
    pltpu.sync_copy(w1r.at[pl.ds(tb, TPW)], wa)
    pltpu.sync_copy(w2r.at[pl.ds(tb, TPW)], wb)
    rb = (rb0, rb1)
    da = (da0, da1, da2)
    db = (db0, db1, db2)
    lsem = (sl0, sl1)
    asem = (sa0, sa1)
    bsem = (sb0, sb1)
    wasem = (swa0, swa1)
    wbsem = (swb0, swb1)
    dasem = (sda0, sda1, sda2)
    dbsem = (sdb0, sdb1, sdb2)
    hl = [None, None]
    hA = [None, None]
    hB = [None, None]
    hWa = [None, None]
    hWb = [None, None]
    hDa = [None, None, None]
    hDb = [None, None, None]

    for c in range(min(3, nch)):
        hDa[c] = pltpu.async_copy(d0r.at[pl.ds(tb + c * CH, CH)], da[c],
                                  dasem[c])
        hDb[c] = pltpu.async_copy(d1r.at[pl.ds(tb + c * CH, CH)], db[c],
                                  dbsem[c])
    hl[0] = pltpu.async_copy(xfr.at[pl.ds(tb, CH)], rb[0], lsem[0])

    for c in range(nch):
        p = c & 1
        m = c % 3
        o = c * CH
        hl[p].wait()
        hDa[m].wait()
        hDb[m].wait()
        hA[p] = pltpu.async_copy(rb[p], xg.at[da[m]], asem[p])
        hB[p] = pltpu.async_copy(rb[p], xg.at[db[m]], bsem[p])
        hWa[p] = pltpu.async_copy(wa.at[pl.ds(o, CH)], wslot.at[da[m]],
                                  wasem[p])
        hWb[p] = pltpu.async_copy(wb.at[pl.ds(o, CH)], wslot.at[db[m]],
                                  wbsem[p])
        if c + 1 < nch:
            q = 1 - p
            if c >= 1:
                hA[q].wait()
                hB[q].wait()
                hWa[q].wait()
                hWb[q].wait()
                if c + 2 < nch:
                    m2 = (c + 2) % 3
                    hDa[m2] = pltpu.async_copy(
                        d0r.at[pl.ds(tb + (c + 2) * CH, CH)], da[m2],
                        dasem[m2])
                    hDb[m2] = pltpu.async_copy(
                        d1r.at[pl.ds(tb + (c + 2) * CH, CH)], db[m2],
                        dbsem[m2])
            hl[q] = pltpu.async_copy(xfr.at[pl.ds(tb + o + CH, CH)], rb[q],
                                     lsem[q])
    for p in (0, 1):
        if nch > p:
            hA[p].wait()
            hB[p].wait()
            hWa[p].wait()
            hWb[p].wait()


def _sc_dispatch(xf, d0, d1, w1, w2):
    mesh = plsc.VectorSubcoreMesh(core_axis_name="c", subcore_axis_name="s")
    fn = pl.kernel(
        _sc_dispatch_body,
        out_type=[
            jax.ShapeDtypeStruct((NPAD, DIM), jnp.float32),
            jax.ShapeDtypeStruct((NPAD,), jnp.float32),
        ],
        mesh=mesh,
        scratch_types=(
            [pltpu.VMEM((CH, DIM), jnp.float32)] * 2
            + [pltpu.VMEM((CH,), jnp.int32)] * 6
            + [pltpu.VMEM((TPW,), jnp.float32)] * 2
            + [pltpu.SemaphoreType.DMA] * 16
        ),
    )
    return fn(xf, d0, d1, w1, w2)


def _ffn_body(be_ref, xg_ref, w1_ref, w3_ref, w2_ref, ws_ref, prev_ref,
              o_ref, first):
    e = be_ref[pl.program_id(0)]

    @pl.when(e >= 0)
    def _():
        xb = xg_ref[...]
        h1 = lax.dot_general(xb, w1_ref[0], (((1,), (1,)), ((), ())),
                             preferred_element_type=jnp.float32)
        h3 = lax.dot_general(xb, w3_ref[0], (((1,), (1,)), ((), ())),
                             preferred_element_type=jnp.float32)
        h = h1 * jax.nn.sigmoid(h1) * h3
        y = lax.dot_general(h, w2_ref[0], (((1,), (1,)), ((), ())),
                            preferred_element_type=jnp.float32)
        if first:
            o_ref[...] = y * ws_ref[...]
        else:
            o_ref[...] = y * ws_ref[...] + prev_ref[...]


def _ffn_half(block_expert, xg, W1h, W3h, W2h, wslot2d, prev, k, interpret=False):
    hid2 = HID // 2
    first = prev is None
    in_specs = [
        pl.BlockSpec((BLKG, DIM), lambda j, be: (j, 0)),
        pl.BlockSpec((1, hid2, DIM),
                     lambda j, be: (jnp.maximum(be[j], 0), k, 0)),
        pl.BlockSpec((1, hid2, DIM),
                     lambda j, be: (jnp.maximum(be[j], 0), k, 0)),
        pl.BlockSpec((1, DIM, hid2),
                     lambda j, be: (jnp.maximum(be[j], 0), 0, k)),
        pl.BlockSpec((BLKG, 1), lambda j, be: (j, 0)),
    ]
    args = [block_expert, xg, W1h, W3h, W2h, wslot2d]
    if first:
        def body(be, xgr, w1r, w3r, w2r, wsr, o):
            _ffn_body(be, xgr, w1r, w3r, w2r, wsr, None, o, True)
    else:
        def body(be, xgr, w1r, w3r, w2r, wsr, prevr, o):
            _ffn_body(be, xgr, w1r, w3r, w2r, wsr, prevr, o, False)
        in_specs.append(pl.BlockSpec((BLKG, DIM), lambda j, be: (j, 0)))
        args.append(prev)
    grid_spec = pltpu.PrefetchScalarGridSpec(
        num_scalar_prefetch=1,
        grid=(NBG1,),
        in_specs=in_specs,
        out_specs=pl.BlockSpec((BLKG, DIM), lambda j, be: (j, 0)),
    )
    return pl.pallas_call(
        body,
        grid_spec=grid_spec,
        out_shape=jax.ShapeDtypeStruct((NPAD, DIM), jnp.float32),
        compiler_params=pltpu.CompilerParams(
            dimension_semantics=("arbitrary",)),
        interpret=interpret,
    )(*args)


def _ffn(block_expert, xg, W1, W3, W2, wslot2d, interpret=False):
    p0 = _ffn_half(block_expert, xg, W1, W3, W2, wslot2d, None, 0, interpret)
    return _ffn_half(block_expert, xg, W1, W3, W2, wslot2d, p0, 1, interpret)


def _ffns_body(xf_ref, w1_ref, w3_ref, w2_ref, o_ref):
    xb = xf_ref[...]
    h1 = lax.dot_general(xb, w1_ref[...], (((1,), (1,)), ((), ())),
                         preferred_element_type=jnp.float32)
    h3 = lax.dot_general(xb, w3_ref[...], (((1,), (1,)), ((), ())),
                         preferred_element_type=jnp.float32)
    h = (h1 * jax.nn.sigmoid(h1) * h3).astype(jnp.bfloat16)
    o_ref[...] = lax.dot_general(h, w2_ref[...], (((1,), (1,)), ((), ())),
                                 preferred_element_type=jnp.float32)


def _ffns(xf, W1sb, W3sb, W2sb, interpret=False):
    nb = N // BLKG
    return pl.pallas_call(
        _ffns_body,
        grid=(nb,),
        in_specs=[
            pl.BlockSpec((BLKG, DIM), lambda j: (j, 0)),
            pl.BlockSpec((HID, DIM), lambda j: (0, 0)),
            pl.BlockSpec((HID, DIM), lambda j: (0, 0)),
            pl.BlockSpec((DIM, HID), lambda j: (0, 0)),
        ],
        out_specs=pl.BlockSpec((BLKG, DIM), lambda j: (j, 0)),
        out_shape=jax.ShapeDtypeStruct((N, DIM), jnp.float32),
        compiler_params=pltpu.CompilerParams(
            dimension_semantics=("arbitrary",)),
        interpret=interpret,
    )(xf, W1sb, W3sb, W2sb)


def _sc_gather_body(outg, d0r, d1r, g0, g1,
                    rb0, rb1, di0, di1, di2,
                    sg0, sg1, sw0, sw1, sd0, sd1, sd2):
    wid = lax.axis_index("s") * 2 + lax.axis_index("c")
    tb = wid * TPW
    nt = 2 * (TPW // CH)
    rb = (rb0, rb1)
    di = (di0, di1, di2)
    sg = (sg0, sg1)
    sw = (sw0, sw1)
    sd = (sd0, sd1, sd2)
    hG = [None, None]
    hW = [None, None]
    hD = [None, None, None]

    def idx_src(t):
        r = d0r if t % 2 == 0 else d1r
        return r.at[pl.ds(tb + (t >> 1) * CH, CH)]

    def out_dst(t):
        r = g0 if t % 2 == 0 else g1
        return r.at[pl.ds(tb + (t >> 1) * CH, CH)]

    for t in range(min(3, nt)):
        hD[t] = pltpu.async_copy(idx_src(t), di[t], sd[t])

    for t in range(nt):
        p = t & 1
        m = t % 3
        if t >= 2:
            hW[p].wait()
        hD[m].wait()
        hG[p] = pltpu.async_copy(outg.at[di[m]], rb[p], sg[p])
        if t >= 1:
            q = 1 - p
            hG[q].wait()
            if t + 2 < nt:
                m2 = (t + 2) % 3
                hD[m2] = pltpu.async_copy(idx_src(t + 2), di[m2], sd[m2])
            hW[q] = pltpu.async_copy(rb[q], out_dst(t - 1), sw[q])
    p = (nt - 1) & 1
    hG[p].wait()
    hW[p] = pltpu.async_copy(rb[p], out_dst(nt - 1), sw[p])
    hW[0].wait()
    hW[1].wait()


def _sc_gather(outg, d0r, d1r):
    mesh = plsc.VectorSubcoreMesh(core_axis_name="c", subcore_axis_name="s")
    fn = pl.kernel(
        _sc_gather_body,
        out_type=[
            jax.ShapeDtypeStruct((N, DIM), jnp.float32),
            jax.ShapeDtypeStruct((N, DIM), jnp.float32),
        ],
        mesh=mesh,
        scratch_types=(
            [pltpu.VMEM((CH, DIM), jnp.float32)] * 2
            + [pltpu.VMEM((CH,), jnp.int32)] * 3
            + [pltpu.SemaphoreType.DMA] * 7
        ),
    )
    return fn(outg, d0r, d1r)


def _combine_body(g0_ref, g1_ref, sh_ref, o_ref):
    o_ref[...] = g0_ref[...] + g1_ref[...] + sh_ref[...]


def _combine(g0, g1, outs, interpret=False):
    blk = min(512, N)
    nb = N // blk
    return pl.pallas_call(
        _combine_body,
        grid=(nb,),
        in_specs=[
            pl.BlockSpec((blk, DIM), lambda i: (i, 0)),
            pl.BlockSpec((blk, DIM), lambda i: (i, 0)),
            pl.BlockSpec((blk, DIM), lambda i: (i, 0)),
        ],
        out_specs=pl.BlockSpec((blk, DIM), lambda i: (i, 0)),
        out_shape=jax.ShapeDtypeStruct((N, DIM), jnp.float32),
        compiler_params=pltpu.CompilerParams(
            dimension_semantics=("parallel",)),
        interpret=interpret,
    )(g0, g1, outs)


def _metadata(counts):
    cnt_pad = ((counts + BLKG - 1) // BLKG) * BLKG
    padbase = jnp.concatenate(
        [jnp.zeros((1,), jnp.int32), jnp.cumsum(cnt_pad)[:-1].astype(jnp.int32),
         jnp.zeros((16 - E,), jnp.int32)])
    off = jnp.arange(NBG1, dtype=jnp.int32) * BLKG
    be1 = (jnp.sum((padbase[None, :E] <= off[:, None]).astype(jnp.int32),
                   axis=1) - 1).astype(jnp.int32)
    total_pad = jnp.sum(cnt_pad)
    block_expert = jnp.where(off < total_pad, be1, -1)
    return padbase, block_expert


def kernel(x, Wg, W1, W2, W3, W1s, W2s, W3s):
    bsz, seqlen, dim = x.shape
    xf = x.reshape(-1, dim)
    ar = jnp.arange(2 * BLKR, dtype=jnp.int32)
    tri = (ar[:, None] > ar[None, :]).astype(jnp.int8)
    i1, i2, r1, r2, w1, w2, cnt = _router(xf, Wg, tri)
    counts = cnt[0]
    padbase, block_expert = _metadata(counts)

    d0, d1 = _dest(i1, i2, r1, r2, padbase.reshape(1, 16))
    dest0 = d0.reshape(N)
    dest1 = d1.reshape(N)
    xg, wslot = _sc_dispatch(xf, dest0, dest1, w1.reshape(N), w2.reshape(N))

    outs = _ffns(xf, W1s, W3s, W2s)
    outg = _ffn(block_expert, xg, W1, W3, W2, wslot.reshape(NPAD, 1))

    g0, g1 = _sc_gather(outg, dest0, dest1)
    y = _combine(g0, g1, outs)
    return y.reshape(bsz, seqlen, dim)
